# baseline (speedup 1.0000x reference)
import functools

import jax
import jax.numpy as jnp
from jax import lax
from jax.experimental import pallas as pl
from jax.experimental.pallas import tpu as pltpu
from jax.experimental.pallas import tpu_sc as plsc

CHUNK = 128
LANES = 16
W8 = 8


def kernel(word, word_table, subword_table, word_to_subwords):
    B = word.shape[0]
    V, D = word_table.shape
    SV = subword_table.shape[0]
    MAX_SUB = word_to_subwords.shape[1]

    info = plsc.get_sparse_core_info()
    NC, NS = info.num_cores, info.num_subcores
    NW = NC * NS
    assert B % (NW * CHUNK) == 0 and D % LANES == 0
    assert (V * MAX_SUB) % W8 == 0
    b_per_w = B // NW
    n_chunks = b_per_w // CHUNK

    pad_idx = SV
    stab_pad = jnp.concatenate(
        [subword_table, jnp.zeros((8, D), subword_table.dtype)], axis=0)
    word2d = word.astype(jnp.int32).reshape(B // CHUNK, CHUNK)
    w2s8 = word_to_subwords.reshape(-1, W8)
    n_rows8 = (V * MAX_SUB) // W8

    mesh = plsc.VectorSubcoreMesh(core_axis_name="c", subcore_axis_name="s")

    @functools.partial(
        pl.kernel,
        out_type=jax.ShapeDtypeStruct((B, D), jnp.float32),
        mesh=mesh,
        compiler_params=pltpu.CompilerParams(
            needs_layout_passes=False, use_tc_tiling_on_sc=False),
        scratch_types=[
            pltpu.VMEM((n_chunks, CHUNK), jnp.int32),
            pltpu.VMEM((2, CHUNK), jnp.int32),
            pltpu.VMEM((2, CHUNK, W8), jnp.int32),
            pltpu.VMEM((MAX_SUB, CHUNK), jnp.int32),
            pltpu.VMEM((MAX_SUB, CHUNK, D), jnp.float32),
            pltpu.VMEM((b_per_w, D), jnp.float32),
            pltpu.SemaphoreType.DMA,
            pltpu.SemaphoreType.DMA,
        ],
    )
    def run(word_hbm, wtab_hbm, stab_hbm, w2s_hbm, out_hbm,
            idx_v, rows_v, ids8_v, subidsT_v, subrows_v, acc_v, sem, sem2):
        wid = lax.axis_index("s") * NC + lax.axis_index("c")
        pltpu.sync_copy(word_hbm.at[pl.ds(wid * n_chunks, n_chunks)], idx_v)

        five = jnp.full((LANES,), MAX_SUB, jnp.int32)
        seven = jnp.full((LANES,), W8 - 1, jnp.int32)
        one = jnp.full((LANES,), 1, jnp.int32)
        last_row = jnp.full((LANES,), n_rows8 - 1, jnp.int32)
        zero16 = jnp.zeros((LANES,), jnp.int32)
        pad16 = jnp.full((LANES,), pad_idx, jnp.int32)

        for c in range(n_chunks):
            dma_w = pltpu.async_copy(
                wtab_hbm.at[idx_v.at[c]],
                acc_v.at[pl.ds(c * CHUNK, CHUNK)], sem)

            for g in range(CHUNK // LANES):
                iv = idx_v[c, pl.ds(g * LANES, LANES)]
                r0 = (iv * five) >> 3
                r1 = jnp.minimum(r0 + one, last_row)
                rows_v[0, pl.ds(g * LANES, LANES)] = r0
                rows_v[1, pl.ds(g * LANES, LANES)] = r1
            d0 = pltpu.async_copy(w2s_hbm.at[rows_v.at[0]], ids8_v.at[0], sem2)
            d1 = pltpu.async_copy(w2s_hbm.at[rows_v.at[1]], ids8_v.at[1], sem2)
            d0.wait()
            d1.wait()

            for j in range(MAX_SUB):
                jj = jnp.full((LANES,), j, jnp.int32)
                for g in range(CHUNK // LANES):
                    iv = idx_v[c, pl.ds(g * LANES, LANES)]
                    pos = ((iv * five) & seven) + jj
                    sel = pos >> 3
                    colw = pos & seven
                    rows = g * LANES + lax.iota(jnp.int32, LANES)
                    v = plsc.load_gather(ids8_v, [sel, rows, colw])
                    v = jnp.where(v == zero16, pad16, v)
                    subidsT_v[j, pl.ds(g * LANES, LANES)] = v

            sub_dmas = [
                pltpu.async_copy(
                    stab_hbm.at[subidsT_v.at[j]], subrows_v.at[j], sem2)
                for j in range(MAX_SUB)
            ]
            dma_w.wait()
            for d in sub_dmas:
                d.wait()

            def body(i, carry):
                for g in range(D // LANES):
                    sl = pl.ds(g * LANES, LANES)
                    s = subrows_v[0, i, sl]
                    for j in range(1, MAX_SUB):
                        s = s + subrows_v[j, i, sl]
                    plsc.addupdate(acc_v.at[c * CHUNK + i, sl], s)
                return carry

            lax.fori_loop(0, CHUNK, body, 0)

        pltpu.sync_copy(acc_v, out_hbm.at[pl.ds(wid * b_per_w, b_per_w)])

    return run(word2d, word_table, stab_pad, w2s8)


# speedup vs baseline: 1.1435x; 1.1435x over previous
import functools

import jax
import jax.numpy as jnp
from jax import lax
from jax.experimental import pallas as pl
from jax.experimental.pallas import tpu as pltpu
from jax.experimental.pallas import tpu_sc as plsc

CHUNK = 128
LANES = 16
W8 = 8


def kernel(word, word_table, subword_table, word_to_subwords):
    B = word.shape[0]
    V, D = word_table.shape
    SV = subword_table.shape[0]
    MAX_SUB = word_to_subwords.shape[1]

    info = plsc.get_sparse_core_info()
    NC, NS = info.num_cores, info.num_subcores
    NW = NC * NS
    assert B % (NW * CHUNK) == 0 and D % LANES == 0
    assert V % W8 == 0
    b_per_w = B // NW
    n_chunks = b_per_w // CHUNK

    pad_idx = SV
    stab_pad = jnp.concatenate(
        [subword_table, jnp.zeros((8, D), subword_table.dtype)], axis=0)
    word2d = word.astype(jnp.int32).reshape(B // CHUNK, CHUNK)
    w2s8 = word_to_subwords.T.reshape(-1, W8)
    rows_per_slot = V // W8

    mesh = plsc.VectorSubcoreMesh(core_axis_name="c", subcore_axis_name="s")

    @functools.partial(
        pl.kernel,
        out_type=jax.ShapeDtypeStruct((B, D), jnp.float32),
        mesh=mesh,
        compiler_params=pltpu.CompilerParams(
            needs_layout_passes=False, use_tc_tiling_on_sc=False),
        scratch_types=[
            pltpu.VMEM((n_chunks, CHUNK), jnp.int32),
            pltpu.VMEM((MAX_SUB, CHUNK), jnp.int32),
            pltpu.VMEM((MAX_SUB, CHUNK, W8), jnp.int32),
            pltpu.VMEM((MAX_SUB, CHUNK), jnp.int32),
            pltpu.VMEM((MAX_SUB, CHUNK, D), jnp.float32),
            pltpu.VMEM((b_per_w, D), jnp.float32),
            pltpu.SemaphoreType.DMA,
            pltpu.SemaphoreType.DMA,
        ],
    )
    def run(word_hbm, wtab_hbm, stab_hbm, w2s_hbm, out_hbm,
            idx_v, rows_v, ids8_v, subidsT_v, subrows_v, acc_v, sem, sem2):
        wid = lax.axis_index("s") * NC + lax.axis_index("c")
        pltpu.sync_copy(word_hbm.at[pl.ds(wid * n_chunks, n_chunks)], idx_v)

        seven = jnp.full((LANES,), W8 - 1, jnp.int32)
        zero16 = jnp.zeros((LANES,), jnp.int32)
        pad16 = jnp.full((LANES,), pad_idx, jnp.int32)

        for c in range(n_chunks):
            dma_w = pltpu.async_copy(
                wtab_hbm.at[idx_v.at[c]],
                acc_v.at[pl.ds(c * CHUNK, CHUNK)], sem)

            for j in range(MAX_SUB):
                base = jnp.full((LANES,), j * rows_per_slot, jnp.int32)
                for g in range(CHUNK // LANES):
                    iv = idx_v[c, pl.ds(g * LANES, LANES)]
                    rows_v[j, pl.ds(g * LANES, LANES)] = base + (iv >> 3)
            id_dmas = [
                pltpu.async_copy(w2s_hbm.at[rows_v.at[j]], ids8_v.at[j], sem2)
                for j in range(MAX_SUB)
            ]
            for d in id_dmas:
                d.wait()

            for j in range(MAX_SUB):
                jj = jnp.full((LANES,), j, jnp.int32)
                for g in range(CHUNK // LANES):
                    iv = idx_v[c, pl.ds(g * LANES, LANES)]
                    rows = g * LANES + lax.iota(jnp.int32, LANES)
                    v = plsc.load_gather(ids8_v, [jj, rows, iv & seven])
                    v = jnp.where(v == zero16, pad16, v)
                    subidsT_v[j, pl.ds(g * LANES, LANES)] = v

            sub_dmas = [
                pltpu.async_copy(
                    stab_hbm.at[subidsT_v.at[j]], subrows_v.at[j], sem2)
                for j in range(MAX_SUB)
            ]
            dma_w.wait()
            for d in sub_dmas:
                d.wait()

            def body(i, carry):
                for g in range(D // LANES):
                    sl = pl.ds(g * LANES, LANES)
                    s = subrows_v[0, i, sl]
                    for j in range(1, MAX_SUB):
                        s = s + subrows_v[j, i, sl]
                    plsc.addupdate(acc_v.at[c * CHUNK + i, sl], s)
                return carry

            lax.fori_loop(0, CHUNK, body, 0)

        pltpu.sync_copy(acc_v, out_hbm.at[pl.ds(wid * b_per_w, b_per_w)])

    return run(word2d, word_table, stab_pad, w2s8)


# speedup vs baseline: 1.6167x; 1.4137x over previous
import functools

import jax
import jax.numpy as jnp
from jax import lax
from jax.experimental import pallas as pl
from jax.experimental.pallas import tpu as pltpu
from jax.experimental.pallas import tpu_sc as plsc

CHUNK = 128
LANES = 16
W8 = 8
D2 = 128


def kernel(word, word_table, subword_table, word_to_subwords):
    B = word.shape[0]
    V, D = word_table.shape
    SV = subword_table.shape[0]
    MAX_SUB = word_to_subwords.shape[1]

    info = plsc.get_sparse_core_info()
    NC, NS = info.num_cores, info.num_subcores
    NW = NC * NS
    assert B % (NW * CHUNK) == 0 and D % LANES == 0
    assert V % W8 == 0
    b_per_w = B // NW
    n_chunks = b_per_w // CHUNK

    pad_idx = SV
    stab_pad = jnp.concatenate(
        [subword_table, jnp.zeros((8, D), subword_table.dtype)], axis=0)
    word2d = word.astype(jnp.int32).reshape(B // CHUNK, CHUNK)
    wt128 = jnp.pad(word_table, ((0, 0), (0, D2 - D)))
    VL = ((V + 127) // 128) * 128
    w2s8 = (
        jnp.pad(word_to_subwords, ((0, VL - V), (0, W8 - MAX_SUB)))
        .reshape(VL // 128, 128, W8)
        .transpose(0, 2, 1)
        .reshape(-1, W8)
    )

    mesh = plsc.VectorSubcoreMesh(core_axis_name="c", subcore_axis_name="s")

    @functools.partial(
        pl.kernel,
        out_type=jax.ShapeDtypeStruct((B, D), jnp.float32),
        mesh=mesh,
        compiler_params=pltpu.CompilerParams(
            needs_layout_passes=False, use_tc_tiling_on_sc=False),
        scratch_types=[
            pltpu.VMEM((n_chunks, CHUNK), jnp.int32),
            pltpu.VMEM((MAX_SUB, CHUNK), jnp.int32),
            pltpu.VMEM((MAX_SUB, CHUNK, W8), jnp.int32),
            pltpu.VMEM((MAX_SUB, CHUNK), jnp.int32),
            pltpu.VMEM((CHUNK, D2), jnp.float32),
            pltpu.VMEM((MAX_SUB, CHUNK, D), jnp.float32),
            pltpu.VMEM((b_per_w, D), jnp.float32),
            pltpu.SemaphoreType.DMA,
            pltpu.SemaphoreType.DMA,
        ],
    )
    def run(word_hbm, wtab_hbm, stab_hbm, w2s_hbm, out_hbm,
            idx_v, rows_v, ids8_v, subidsT_v, wrow_v, subrows_v, acc_v,
            sem, sem2):
        wid = lax.axis_index("s") * NC + lax.axis_index("c")
        pltpu.sync_copy(word_hbm.at[pl.ds(wid * n_chunks, n_chunks)], idx_v)

        seven = jnp.full((LANES,), W8 - 1, jnp.int32)
        fifteen = jnp.full((LANES,), 15, jnp.int32)
        zero16 = jnp.zeros((LANES,), jnp.int32)
        pad16 = jnp.full((LANES,), pad_idx, jnp.int32)

        for c in range(n_chunks):
            dma_w = pltpu.async_copy(
                wtab_hbm.at[idx_v.at[c]], wrow_v, sem)

            for j in range(MAX_SUB):
                joff = jnp.full((LANES,), j * (128 // W8), jnp.int32)
                for g in range(CHUNK // LANES):
                    iv = idx_v[c, pl.ds(g * LANES, LANES)]
                    r = ((iv >> 7) << 7) + joff + ((iv >> 3) & fifteen)
                    rows_v[j, pl.ds(g * LANES, LANES)] = r
            id_dmas = [
                pltpu.async_copy(w2s_hbm.at[rows_v.at[j]], ids8_v.at[j], sem2)
                for j in range(MAX_SUB)
            ]
            for d in id_dmas:
                d.wait()

            for j in range(MAX_SUB):
                jj = jnp.full((LANES,), j, jnp.int32)
                for g in range(CHUNK // LANES):
                    iv = idx_v[c, pl.ds(g * LANES, LANES)]
                    rows = g * LANES + lax.iota(jnp.int32, LANES)
                    v = plsc.load_gather(ids8_v, [jj, rows, iv & seven])
                    v = jnp.where(v == zero16, pad16, v)
                    subidsT_v[j, pl.ds(g * LANES, LANES)] = v

            sub_dmas = [
                pltpu.async_copy(
                    stab_hbm.at[subidsT_v.at[j]], subrows_v.at[j], sem2)
                for j in range(MAX_SUB)
            ]
            dma_w.wait()
            for d in sub_dmas:
                d.wait()

            def body(i, carry):
                for g in range(D // LANES):
                    sl = pl.ds(g * LANES, LANES)
                    s = wrow_v[i, sl]
                    for j in range(MAX_SUB):
                        s = s + subrows_v[j, i, sl]
                    acc_v[c * CHUNK + i, sl] = s
                return carry

            lax.fori_loop(0, CHUNK, body, 0)

        pltpu.sync_copy(acc_v, out_hbm.at[pl.ds(wid * b_per_w, b_per_w)])

    return run(word2d, wt128, stab_pad, w2s8)
